# baseline (speedup 1.0000x reference)
import functools

import jax
import jax.numpy as jnp
from jax import lax
from jax.experimental import pallas as pl
from jax.experimental.pallas import tpu as pltpu
from jax.experimental.pallas import tpu_sc as plsc

NC = 2
NS = 16
NW = NC * NS

B = 4096
L = 50
DIM = 32
NT = 4
CB = B // NW
VL = 16
VOCABS = (1000000, 1000000, 100000, 100000)


def _tr_kernel(tt0, tt1, tt2, tt3, tl0, tl1, tl2, tl3, o0, o1, o2, o3,
               slab0, slab1, comb0, comb1, si0, si1, so0, so1):
    wid = lax.axis_index("s") * NC + lax.axis_index("c")
    slabs = (slab0, slab1)
    combs = (comb0, comb1)
    sin = (si0, si1)
    sout = (so0, so1)
    iota = lax.iota(jnp.int32, VL)
    idxc = [iota + u * VL for u in range(DIM // VL)]

    def transpose_rows(slab, comb, nrows):
        def rowbody(i, c):
            si = jnp.full((VL,), 0, jnp.int32) + i
            for u in range(DIM // VL):
                v = plsc.load_gather(slab, [idxc[u], si])
                comb[pl.ds(i * DIM + u * VL, VL)] = v
            return c
        lax.fori_loop(0, nrows, rowbody, 0)

    for t, (tt, tl, out) in enumerate(zip((tt0, tt1, tt2, tt3),
                                          (tl0, tl1, tl2, tl3),
                                          (o0, o1, o2, o3))):
        v = VOCABS[t]
        nfull = v // CB
        tail = v % CB
        nj = (nfull - wid + NW - 1) // NW

        def issue_in(k, p, tt=tt, nj=nj):
            @pl.when(k < nj)
            def _():
                blk = wid + k * NW
                pltpu.async_copy(tt.at[:, pl.ds(blk * CB, CB)],
                                 slabs[p], sin[p])

        def wait_in(p, tt=tt):
            pltpu.make_async_copy(tt.at[:, pl.ds(0, CB)], slabs[p],
                                  sin[p]).wait()

        def issue_out(k, p, out=out):
            blk = wid + k * NW
            pltpu.async_copy(combs[p],
                             out.at[pl.ds(blk * CB * DIM, CB * DIM)],
                             sout[p])

        def wait_out(p, out=out):
            pltpu.make_async_copy(out.at[pl.ds(0, CB * DIM)], combs[p],
                                  sout[p]).wait()

        issue_in(0, 0)
        issue_in(1, 1)

        def pairbody(k2, c, nj=nj):
            for p in range(2):
                k = k2 * 2 + p

                @pl.when(k < nj)
                def _(k=k, p=p):
                    wait_in(p)

                    @pl.when(k >= 2)
                    def _():
                        wait_out(p)
                    transpose_rows(slabs[p], combs[p], CB)
                    issue_out(k, p)
                    issue_in(k + 2, p)
            return c

        lax.fori_loop(0, (nj + 1) // 2, pairbody, 0)

        @pl.when(nj >= 1)
        def _():
            wait_out(0)

        @pl.when(nj >= 2)
        def _():
            wait_out(1)

        if tail:
            owner = nfull % NW

            @pl.when(wid == owner)
            def _(tl=tl, out=out, nfull=nfull, tail=tail):
                pltpu.sync_copy(
                    tl, out.at[pl.ds(nfull * CB * DIM, tail * DIM)])


@jax.jit
def _tr_run(tt0, tt1, tt2, tt3, tl0, tl1, tl2, tl3):
    mesh = plsc.VectorSubcoreMesh(core_axis_name="c", subcore_axis_name="s",
                                  num_cores=NC, num_subcores=NS)
    k = pl.kernel(
        _tr_kernel,
        out_type=tuple(jax.ShapeDtypeStruct((v * DIM,), jnp.float32)
                       for v in VOCABS),
        mesh=mesh,
        scratch_types=(
            [pltpu.VMEM((DIM, CB), jnp.float32)] * 2
            + [pltpu.VMEM((CB * DIM,), jnp.float32)] * 2
            + [pltpu.SemaphoreType.DMA] * 4
        ),
        compiler_params=pltpu.CompilerParams(use_tc_tiling_on_sc=True,
                                             needs_layout_passes=False),
    )
    return k(tt0, tt1, tt2, tt3, tl0, tl1, tl2, tl3)


def _emb_kernel(ids0, ids1, ids2, ids3, t0, t1, t2, t3, out_hbm,
                idx_v, rows_v, sem):
    wid = lax.axis_index("s") * NC + lax.axis_index("c")
    b0 = wid * CB
    tables = (t0, t1, t2, t3)
    ids = (ids0, ids1, ids2, ids3)
    for t in range(NT):
        pltpu.sync_copy(ids[t].at[:, pl.ds(b0, CB)], idx_v)

        def body(l, carry, t=t):
            pltpu.async_copy(tables[t].at[idx_v.at[l]], rows_v, sem).wait()
            pltpu.sync_copy(
                rows_v,
                out_hbm.at[l, pl.ds(b0, CB), pl.ds(t * DIM, DIM)])
            return carry

        lax.fori_loop(0, L, body, 0)


@jax.jit
def _run(ids0, ids1, ids2, ids3, t0, t1, t2, t3):
    mesh = plsc.VectorSubcoreMesh(core_axis_name="c", subcore_axis_name="s",
                                  num_cores=NC, num_subcores=NS)
    k = pl.kernel(
        _emb_kernel,
        out_type=jax.ShapeDtypeStruct((L, B, NT * DIM), jnp.float32),
        mesh=mesh,
        scratch_types=[
            pltpu.VMEM((L, CB), jnp.int32),
            pltpu.VMEM((CB, DIM), jnp.float32),
            pltpu.SemaphoreType.DMA,
        ],
        compiler_params=pltpu.CompilerParams(use_tc_tiling_on_sc=False),
    )
    return k(ids0, ids1, ids2, ids3, t0, t1, t2, t3)


def kernel(ids_0, ids_1, ids_2, ids_3, table_0, table_1, table_2, table_3):
    tabs = (table_0, table_1, table_2, table_3)
    tails = [tb[(v // CB) * CB:, :].reshape(-1)
             for tb, v in zip(tabs, VOCABS)]
    dense = _tr_run(table_0.T, table_1.T, table_2.T, table_3.T, *tails)
    tables = [d.reshape(v, DIM) for d, v in zip(dense, VOCABS)]
    ids = [i.astype(jnp.int32).T for i in (ids_0, ids_1, ids_2, ids_3)]
    out = _run(*ids, *tables)
    return out.transpose(1, 0, 2)


# speedup vs baseline: 1.1147x; 1.1147x over previous
import functools

import jax
import jax.numpy as jnp
from jax import lax
from jax.experimental import pallas as pl
from jax.experimental.pallas import tpu as pltpu
from jax.experimental.pallas import tpu_sc as plsc

NC = 2
NS = 16
NW = NC * NS

B = 4096
L = 50
DIM = 32
NT = 4
CB = B // NW
VL = 16
VOCABS = (1000000, 1000000, 100000, 100000)


def _tr_kernel(tt0, tt1, tt2, tt3, tl0, tl1, tl2, tl3, o0, o1, o2, o3,
               slab0, slab1, comb0, comb1, si0, si1, so0, so1):
    wid = lax.axis_index("s") * NC + lax.axis_index("c")
    slabs = (slab0, slab1)
    combs = (comb0, comb1)
    sin = (si0, si1)
    sout = (so0, so1)
    iota = lax.iota(jnp.int32, VL)
    idxc = [iota + u * VL for u in range(DIM // VL)]

    UNROLL = 8

    def transpose_rows(slab, comb, nrows):
        zero = jnp.zeros((VL,), jnp.int32)

        def rowbody(j, base):
            i0 = j * UNROLL
            for r in range(UNROLL):
                si = base + r
                vs = [plsc.load_gather(slab, [idxc[u], si])
                      for u in range(DIM // VL)]
                for u in range(DIM // VL):
                    comb[pl.ds((i0 + r) * DIM + u * VL, VL)] = vs[u]
            return base + UNROLL

        lax.fori_loop(0, nrows // UNROLL, rowbody, zero)

    for t, (tt, tl, out) in enumerate(zip((tt0, tt1, tt2, tt3),
                                          (tl0, tl1, tl2, tl3),
                                          (o0, o1, o2, o3))):
        v = VOCABS[t]
        nfull = v // CB
        tail = v % CB
        nj = (nfull - wid + NW - 1) // NW

        def issue_in(k, p, tt=tt, nj=nj):
            @pl.when(k < nj)
            def _():
                blk = wid + k * NW
                pltpu.async_copy(tt.at[:, pl.ds(blk * CB, CB)],
                                 slabs[p], sin[p])

        def wait_in(p, tt=tt):
            pltpu.make_async_copy(tt.at[:, pl.ds(0, CB)], slabs[p],
                                  sin[p]).wait()

        def issue_out(k, p, out=out):
            blk = wid + k * NW
            pltpu.async_copy(combs[p],
                             out.at[pl.ds(blk * CB * DIM, CB * DIM)],
                             sout[p])

        def wait_out(p, out=out):
            pltpu.make_async_copy(out.at[pl.ds(0, CB * DIM)], combs[p],
                                  sout[p]).wait()

        issue_in(0, 0)
        issue_in(1, 1)

        def pairbody(k2, c, nj=nj):
            for p in range(2):
                k = k2 * 2 + p

                @pl.when(k < nj)
                def _(k=k, p=p):
                    wait_in(p)

                    @pl.when(k >= 2)
                    def _():
                        wait_out(p)
                    transpose_rows(slabs[p], combs[p], CB)
                    issue_out(k, p)
                    issue_in(k + 2, p)
            return c

        lax.fori_loop(0, (nj + 1) // 2, pairbody, 0)

        @pl.when(nj >= 1)
        def _():
            wait_out(0)

        @pl.when(nj >= 2)
        def _():
            wait_out(1)

        if tail:
            owner = nfull % NW

            @pl.when(wid == owner)
            def _(tl=tl, out=out, nfull=nfull, tail=tail):
                pltpu.sync_copy(
                    tl, out.at[pl.ds(nfull * CB * DIM, tail * DIM)])


@jax.jit
def _tr_run(tt0, tt1, tt2, tt3, tl0, tl1, tl2, tl3):
    mesh = plsc.VectorSubcoreMesh(core_axis_name="c", subcore_axis_name="s",
                                  num_cores=NC, num_subcores=NS)
    k = pl.kernel(
        _tr_kernel,
        out_type=tuple(jax.ShapeDtypeStruct((v * DIM,), jnp.float32)
                       for v in VOCABS),
        mesh=mesh,
        scratch_types=(
            [pltpu.VMEM((DIM, CB), jnp.float32)] * 2
            + [pltpu.VMEM((CB * DIM,), jnp.float32)] * 2
            + [pltpu.SemaphoreType.DMA] * 4
        ),
        compiler_params=pltpu.CompilerParams(use_tc_tiling_on_sc=True,
                                             needs_layout_passes=False),
    )
    return k(tt0, tt1, tt2, tt3, tl0, tl1, tl2, tl3)


def _emb_kernel(ids0, ids1, ids2, ids3, t0, t1, t2, t3, out_hbm,
                idx_v, rows_v, sem):
    wid = lax.axis_index("s") * NC + lax.axis_index("c")
    b0 = wid * CB
    tables = (t0, t1, t2, t3)
    ids = (ids0, ids1, ids2, ids3)
    for t in range(NT):
        pltpu.sync_copy(ids[t].at[:, pl.ds(b0, CB)], idx_v)

        def body(l, carry, t=t):
            pltpu.async_copy(tables[t].at[idx_v.at[l]], rows_v, sem).wait()
            pltpu.sync_copy(
                rows_v,
                out_hbm.at[l, pl.ds(b0, CB), pl.ds(t * DIM, DIM)])
            return carry

        lax.fori_loop(0, L, body, 0)


@jax.jit
def _run(ids0, ids1, ids2, ids3, t0, t1, t2, t3):
    mesh = plsc.VectorSubcoreMesh(core_axis_name="c", subcore_axis_name="s",
                                  num_cores=NC, num_subcores=NS)
    k = pl.kernel(
        _emb_kernel,
        out_type=jax.ShapeDtypeStruct((L, B, NT * DIM), jnp.float32),
        mesh=mesh,
        scratch_types=[
            pltpu.VMEM((L, CB), jnp.int32),
            pltpu.VMEM((CB, DIM), jnp.float32),
            pltpu.SemaphoreType.DMA,
        ],
        compiler_params=pltpu.CompilerParams(use_tc_tiling_on_sc=False),
    )
    return k(ids0, ids1, ids2, ids3, t0, t1, t2, t3)


def kernel(ids_0, ids_1, ids_2, ids_3, table_0, table_1, table_2, table_3):
    tabs = (table_0, table_1, table_2, table_3)
    tails = [tb[(v // CB) * CB:, :].reshape(-1)
             for tb, v in zip(tabs, VOCABS)]
    dense = _tr_run(table_0.T, table_1.T, table_2.T, table_3.T, *tails)
    tables = [d.reshape(v, DIM) for d, v in zip(dense, VOCABS)]
    ids = [i.astype(jnp.int32).T for i in (ids_0, ids_1, ids_2, ids_3)]
    out = _run(*ids, *tables)
    return out.transpose(1, 0, 2)


# speedup vs baseline: 2.3620x; 2.1190x over previous
import functools

import jax
import jax.numpy as jnp
from jax import lax
from jax.experimental import pallas as pl
from jax.experimental.pallas import tpu as pltpu
from jax.experimental.pallas import tpu_sc as plsc

NC = 2
NS = 16
NW = NC * NS

B = 4096
L = 50
DIM = 32
NT = 4
CB = B // NW
VL = 16
VOCABS = (1000000, 1000000, 100000, 100000)


def _tr_kernel(tt0, tt1, tt2, tt3, tl0, tl1, tl2, tl3, o0, o1, o2, o3,
               slab0, slab1, comb0, comb1, si0, si1, so0, so1):
    wid = lax.axis_index("s") * NC + lax.axis_index("c")
    slabs = (slab0, slab1)
    combs = (comb0, comb1)
    sin = (si0, si1)
    sout = (so0, so1)
    iota = lax.iota(jnp.int32, VL)
    idxc = [iota + u * VL for u in range(DIM // VL)]

    UNROLL = 8

    def transpose_rows(slab, comb, nrows):
        def rowbody(j, vi):
            for r in range(UNROLL):
                vim = lax.bitwise_and(vi, CB - 1)
                st = lax.shift_left(vim, 5)
                for u in range(DIM // VL):
                    v = plsc.load_gather(slab, [idxc[u], vim])
                    plsc.store_scatter(comb,
                                       [lax.bitwise_or(st, idxc[u])], v)
                vi = vi + 1
            return vi

        lax.fori_loop(0, nrows // UNROLL, rowbody, iota)

    for t, (tt, tl, out) in enumerate(zip((tt0, tt1, tt2, tt3),
                                          (tl0, tl1, tl2, tl3),
                                          (o0, o1, o2, o3))):
        v = VOCABS[t]
        nfull = v // CB
        tail = v % CB
        nj = (nfull - wid + NW - 1) // NW

        def issue_in(k, p, tt=tt, nj=nj):
            @pl.when(k < nj)
            def _():
                blk = wid + k * NW
                pltpu.async_copy(tt.at[:, pl.ds(blk * CB, CB)],
                                 slabs[p], sin[p])

        def wait_in(p, tt=tt):
            pltpu.make_async_copy(tt.at[:, pl.ds(0, CB)], slabs[p],
                                  sin[p]).wait()

        def issue_out(k, p, out=out):
            blk = wid + k * NW
            pltpu.async_copy(combs[p],
                             out.at[pl.ds(blk * CB * DIM, CB * DIM)],
                             sout[p])

        def wait_out(p, out=out):
            pltpu.make_async_copy(out.at[pl.ds(0, CB * DIM)], combs[p],
                                  sout[p]).wait()

        issue_in(0, 0)
        issue_in(1, 1)

        def pairbody(k2, c, nj=nj):
            for p in range(2):
                k = k2 * 2 + p

                @pl.when(k < nj)
                def _(k=k, p=p):
                    wait_in(p)

                    @pl.when(k >= 2)
                    def _():
                        wait_out(p)
                    transpose_rows(slabs[p], combs[p], CB)
                    issue_out(k, p)
                    issue_in(k + 2, p)
            return c

        lax.fori_loop(0, (nj + 1) // 2, pairbody, 0)

        @pl.when(nj >= 1)
        def _():
            wait_out(0)

        @pl.when(nj >= 2)
        def _():
            wait_out(1)

        if tail:
            owner = nfull % NW

            @pl.when(wid == owner)
            def _(tl=tl, out=out, nfull=nfull, tail=tail):
                pltpu.sync_copy(
                    tl, out.at[pl.ds(nfull * CB * DIM, tail * DIM)])


@jax.jit
def _tr_run(tt0, tt1, tt2, tt3, tl0, tl1, tl2, tl3):
    mesh = plsc.VectorSubcoreMesh(core_axis_name="c", subcore_axis_name="s",
                                  num_cores=NC, num_subcores=NS)
    k = pl.kernel(
        _tr_kernel,
        out_type=tuple(jax.ShapeDtypeStruct((v * DIM,), jnp.float32)
                       for v in VOCABS),
        mesh=mesh,
        scratch_types=(
            [pltpu.VMEM((DIM, CB), jnp.float32)] * 2
            + [pltpu.VMEM((CB * DIM,), jnp.float32)] * 2
            + [pltpu.SemaphoreType.DMA] * 4
        ),
        compiler_params=pltpu.CompilerParams(use_tc_tiling_on_sc=True,
                                             needs_layout_passes=False),
    )
    return k(tt0, tt1, tt2, tt3, tl0, tl1, tl2, tl3)


def _emb_kernel(ids0, ids1, ids2, ids3, t0, t1, t2, t3, out_hbm,
                idx_v, rows_v, sem):
    wid = lax.axis_index("s") * NC + lax.axis_index("c")
    b0 = wid * CB
    tables = (t0, t1, t2, t3)
    ids = (ids0, ids1, ids2, ids3)
    for t in range(NT):
        pltpu.sync_copy(ids[t].at[:, pl.ds(b0, CB)], idx_v)

        def body(l, carry, t=t):
            pltpu.async_copy(tables[t].at[idx_v.at[l]], rows_v, sem).wait()
            pltpu.sync_copy(
                rows_v,
                out_hbm.at[l, pl.ds(b0, CB), pl.ds(t * DIM, DIM)])
            return carry

        lax.fori_loop(0, L, body, 0)


@jax.jit
def _run(ids0, ids1, ids2, ids3, t0, t1, t2, t3):
    mesh = plsc.VectorSubcoreMesh(core_axis_name="c", subcore_axis_name="s",
                                  num_cores=NC, num_subcores=NS)
    k = pl.kernel(
        _emb_kernel,
        out_type=jax.ShapeDtypeStruct((L, B, NT * DIM), jnp.float32),
        mesh=mesh,
        scratch_types=[
            pltpu.VMEM((L, CB), jnp.int32),
            pltpu.VMEM((CB, DIM), jnp.float32),
            pltpu.SemaphoreType.DMA,
        ],
        compiler_params=pltpu.CompilerParams(use_tc_tiling_on_sc=False),
    )
    return k(ids0, ids1, ids2, ids3, t0, t1, t2, t3)


def kernel(ids_0, ids_1, ids_2, ids_3, table_0, table_1, table_2, table_3):
    tabs = (table_0, table_1, table_2, table_3)
    tails = [tb[(v // CB) * CB:, :].reshape(-1)
             for tb, v in zip(tabs, VOCABS)]
    dense = _tr_run(table_0.T, table_1.T, table_2.T, table_3.T, *tails)
    tables = [d.reshape(v, DIM) for d, v in zip(dense, VOCABS)]
    ids = [i.astype(jnp.int32).T for i in (ids_0, ids_1, ids_2, ids_3)]
    out = _run(*ids, *tables)
    return out.transpose(1, 0, 2)


# speedup vs baseline: 2.6722x; 1.1313x over previous
import functools

import jax
import jax.numpy as jnp
from jax import lax
from jax.experimental import pallas as pl
from jax.experimental.pallas import tpu as pltpu
from jax.experimental.pallas import tpu_sc as plsc

NC = 2
NS = 16
NW = NC * NS

B = 4096
L = 50
DIM = 32
NT = 4
CB = B // NW
VL = 16
VOCABS = (1000000, 1000000, 100000, 100000)


def _tr_kernel(tt0, tt1, tt2, tt3, tl0, tl1, tl2, tl3, o0, o1, o2, o3,
               slab0, slab1, comb0, comb1, si0, si1, so0, so1):
    wid = lax.axis_index("s") * NC + lax.axis_index("c")
    slabs = (slab0, slab1)
    combs = (comb0, comb1)
    sin = (si0, si1)
    sout = (so0, so1)
    iota = lax.iota(jnp.int32, VL)
    idxc = [iota + u * VL for u in range(DIM // VL)]

    UNROLL = 8

    def transpose_rows(slab, comb, nrows):
        def rowbody(j, vi):
            for r in range(UNROLL):
                vim = lax.bitwise_and(vi, CB - 1)
                st = lax.shift_left(vim, 5)
                for u in range(DIM // VL):
                    v = plsc.load_gather(slab, [idxc[u], vim])
                    plsc.store_scatter(comb,
                                       [lax.bitwise_or(st, idxc[u])], v)
                vi = vi + 1
            return vi

        lax.fori_loop(0, nrows // UNROLL, rowbody, iota)

    for t, (tt, tl, out) in enumerate(zip((tt0, tt1, tt2, tt3),
                                          (tl0, tl1, tl2, tl3),
                                          (o0, o1, o2, o3))):
        v = VOCABS[t]
        nfull = v // CB
        tail = v % CB
        nj = (nfull - wid + NW - 1) // NW

        def issue_in(k, p, tt=tt, nj=nj):
            @pl.when(k < nj)
            def _():
                blk = wid + k * NW
                pltpu.async_copy(tt.at[:, pl.ds(blk * CB, CB)],
                                 slabs[p], sin[p])

        def wait_in(p, tt=tt):
            pltpu.make_async_copy(tt.at[:, pl.ds(0, CB)], slabs[p],
                                  sin[p]).wait()

        def issue_out(k, p, out=out):
            blk = wid + k * NW
            pltpu.async_copy(combs[p],
                             out.at[pl.ds(blk * CB * DIM, CB * DIM)],
                             sout[p])

        def wait_out(p, out=out):
            pltpu.make_async_copy(out.at[pl.ds(0, CB * DIM)], combs[p],
                                  sout[p]).wait()

        issue_in(0, 0)
        issue_in(1, 1)

        def pairbody(k2, c, nj=nj):
            for p in range(2):
                k = k2 * 2 + p

                @pl.when(k < nj)
                def _(k=k, p=p):
                    wait_in(p)

                    @pl.when(k >= 2)
                    def _():
                        wait_out(p)
                    transpose_rows(slabs[p], combs[p], CB)
                    issue_out(k, p)
                    issue_in(k + 2, p)
            return c

        lax.fori_loop(0, (nj + 1) // 2, pairbody, 0)

        @pl.when(nj >= 1)
        def _():
            wait_out(0)

        @pl.when(nj >= 2)
        def _():
            wait_out(1)

        if tail:
            owner = nfull % NW

            @pl.when(wid == owner)
            def _(tl=tl, out=out, nfull=nfull, tail=tail):
                pltpu.sync_copy(
                    tl, out.at[pl.ds(nfull * CB * DIM, tail * DIM)])


@jax.jit
def _tr_run(tt0, tt1, tt2, tt3, tl0, tl1, tl2, tl3):
    mesh = plsc.VectorSubcoreMesh(core_axis_name="c", subcore_axis_name="s",
                                  num_cores=NC, num_subcores=NS)
    k = pl.kernel(
        _tr_kernel,
        out_type=tuple(jax.ShapeDtypeStruct((v * DIM,), jnp.float32)
                       for v in VOCABS),
        mesh=mesh,
        scratch_types=(
            [pltpu.VMEM((DIM, CB), jnp.float32)] * 2
            + [pltpu.VMEM((CB * DIM,), jnp.float32)] * 2
            + [pltpu.SemaphoreType.DMA] * 4
        ),
        compiler_params=pltpu.CompilerParams(use_tc_tiling_on_sc=True,
                                             needs_layout_passes=False),
    )
    return k(tt0, tt1, tt2, tt3, tl0, tl1, tl2, tl3)


def _emb_kernel(ids0, ids1, ids2, ids3, t0, t1, t2, t3, out_hbm,
                idx_v, rows0, rows1, sg0, sg1, sw0, sw1):
    wid = lax.axis_index("s") * NC + lax.axis_index("c")
    b0 = wid * CB
    tables = (t0, t1, t2, t3)
    ids = (ids0, ids1, ids2, ids3)
    rows = (rows0, rows1)
    sg = (sg0, sg1)
    sw = (sw0, sw1)
    for t in range(NT):
        tab = tables[t]
        pltpu.sync_copy(ids[t].at[:, pl.ds(b0, CB)], idx_v)

        def gather(l, p, tab=tab):
            pltpu.async_copy(tab.at[idx_v.at[l]], rows[p], sg[p])

        def wait_gather(p, tab=tab):
            pltpu.make_async_copy(tab.at[idx_v.at[0]], rows[p],
                                  sg[p]).wait()

        def put(l, p, t=t):
            pltpu.async_copy(
                rows[p],
                out_hbm.at[l, pl.ds(b0, CB), pl.ds(t * DIM, DIM)], sw[p])

        def wait_put(p, t=t):
            pltpu.make_async_copy(
                rows[p],
                out_hbm.at[0, pl.ds(b0, CB), pl.ds(t * DIM, DIM)],
                sw[p]).wait()

        gather(0, 0)

        def body(l2, carry):
            for p in range(2):
                l = l2 * 2 + p
                pn = (p + 1) % 2

                @pl.when(l + 1 < L)
                def _(l=l, pn=pn):
                    @pl.when(l >= 1)
                    def _():
                        wait_put(pn)
                    gather(l + 1, pn)
                wait_gather(p)
                put(l, p)
            return carry

        lax.fori_loop(0, L // 2, body, 0)
        wait_put(0)
        wait_put(1)


@jax.jit
def _run(ids0, ids1, ids2, ids3, t0, t1, t2, t3):
    mesh = plsc.VectorSubcoreMesh(core_axis_name="c", subcore_axis_name="s",
                                  num_cores=NC, num_subcores=NS)
    k = pl.kernel(
        _emb_kernel,
        out_type=jax.ShapeDtypeStruct((L, B, NT * DIM), jnp.float32),
        mesh=mesh,
        scratch_types=(
            [pltpu.VMEM((L, CB), jnp.int32)]
            + [pltpu.VMEM((CB, DIM), jnp.float32)] * 2
            + [pltpu.SemaphoreType.DMA] * 4
        ),
        compiler_params=pltpu.CompilerParams(use_tc_tiling_on_sc=False),
    )
    return k(ids0, ids1, ids2, ids3, t0, t1, t2, t3)


def kernel(ids_0, ids_1, ids_2, ids_3, table_0, table_1, table_2, table_3):
    tabs = (table_0, table_1, table_2, table_3)
    tails = [tb[(v // CB) * CB:, :].reshape(-1)
             for tb, v in zip(tabs, VOCABS)]
    dense = _tr_run(table_0.T, table_1.T, table_2.T, table_3.T, *tails)
    tables = [d.reshape(v, DIM) for d, v in zip(dense, VOCABS)]
    ids = [i.astype(jnp.int32).T for i in (ids_0, ids_1, ids_2, ids_3)]
    out = _run(*ids, *tables)
    return out.transpose(1, 0, 2)


# speedup vs baseline: 3.7112x; 1.3888x over previous
import functools

import jax
import jax.numpy as jnp
from jax import lax
from jax.experimental import pallas as pl
from jax.experimental.pallas import tpu as pltpu
from jax.experimental.pallas import tpu_sc as plsc

NC = 2
NS = 16
NW = NC * NS

B = 4096
L = 50
DIM = 32
NT = 4
CB = B // NW
VL = 16
VOCABS = (1000000, 1000000, 100000, 100000)


def _tr_kernel(tt0, tt1, tt2, tt3, tl0, tl1, tl2, tl3, o0, o1, o2, o3,
               slab0, slab1, comb0, comb1, si0, si1, so0, so1):
    wid = lax.axis_index("s") * NC + lax.axis_index("c")
    slabs = (slab0, slab1)
    combs = (comb0, comb1)
    sin = (si0, si1)
    sout = (so0, so1)
    iota = lax.iota(jnp.int32, VL)
    idxc = [iota + u * VL for u in range(DIM // VL)]

    UNROLL = 8

    def transpose_rows(slab, comb, nrows):
        def rowbody(j, vi):
            pairs = []
            for r in range(UNROLL):
                vim = lax.bitwise_and(vi + r, CB - 1)
                st = lax.shift_left(vim, 5)
                for u in range(DIM // VL):
                    v = plsc.load_gather(slab, [idxc[u], vim])
                    pairs.append((lax.bitwise_or(st, idxc[u]), v))
            for sidx, v in pairs:
                plsc.store_scatter(comb, [sidx], v)
            return vi + UNROLL

        lax.fori_loop(0, nrows // UNROLL, rowbody, iota)

    for t, (tt, tl, out) in enumerate(zip((tt0, tt1, tt2, tt3),
                                          (tl0, tl1, tl2, tl3),
                                          (o0, o1, o2, o3))):
        v = VOCABS[t]
        nfull = v // CB
        tail = v % CB
        nj = (nfull - wid + NW - 1) // NW

        def issue_in(k, p, tt=tt, nj=nj):
            @pl.when(k < nj)
            def _():
                blk = wid + k * NW
                pltpu.async_copy(tt.at[:, pl.ds(blk * CB, CB)],
                                 slabs[p], sin[p])

        def wait_in(p, tt=tt):
            pltpu.make_async_copy(tt.at[:, pl.ds(0, CB)], slabs[p],
                                  sin[p]).wait()

        def issue_out(k, p, out=out):
            blk = wid + k * NW
            pltpu.async_copy(combs[p],
                             out.at[pl.ds(blk * CB * DIM, CB * DIM)],
                             sout[p])

        def wait_out(p, out=out):
            pltpu.make_async_copy(out.at[pl.ds(0, CB * DIM)], combs[p],
                                  sout[p]).wait()

        issue_in(0, 0)
        issue_in(1, 1)

        def pairbody(k2, c, nj=nj):
            for p in range(2):
                k = k2 * 2 + p

                @pl.when(k < nj)
                def _(k=k, p=p):
                    wait_in(p)

                    @pl.when(k >= 2)
                    def _():
                        wait_out(p)
                    transpose_rows(slabs[p], combs[p], CB)
                    issue_out(k, p)
                    issue_in(k + 2, p)
            return c

        lax.fori_loop(0, (nj + 1) // 2, pairbody, 0)

        @pl.when(nj >= 1)
        def _():
            wait_out(0)

        @pl.when(nj >= 2)
        def _():
            wait_out(1)

        if tail:
            owner = nfull % NW

            @pl.when(wid == owner)
            def _(tl=tl, out=out, nfull=nfull, tail=tail):
                pltpu.sync_copy(
                    tl, out.at[pl.ds(nfull * CB * DIM, tail * DIM)])


@jax.jit
def _tr_run(tt0, tt1, tt2, tt3, tl0, tl1, tl2, tl3):
    mesh = plsc.VectorSubcoreMesh(core_axis_name="c", subcore_axis_name="s",
                                  num_cores=NC, num_subcores=NS)
    k = pl.kernel(
        _tr_kernel,
        out_type=tuple(jax.ShapeDtypeStruct((v * DIM,), jnp.float32)
                       for v in VOCABS),
        mesh=mesh,
        scratch_types=(
            [pltpu.VMEM((DIM, CB), jnp.float32)] * 2
            + [pltpu.VMEM((CB * DIM,), jnp.float32)] * 2
            + [pltpu.SemaphoreType.DMA] * 4
        ),
        compiler_params=pltpu.CompilerParams(use_tc_tiling_on_sc=True,
                                             needs_layout_passes=False),
    )
    return k(tt0, tt1, tt2, tt3, tl0, tl1, tl2, tl3)


def _emb_kernel(ids0, ids1, ids2, ids3, t0, t1, t2, t3, out_hbm,
                idx_v, rows0, rows1, sg0, sg1, sw0, sw1):
    wid = lax.axis_index("s") * NC + lax.axis_index("c")
    b0 = wid * CB
    tables = (t0, t1, t2, t3)
    ids = (ids0, ids1, ids2, ids3)
    rows = (rows0, rows1)
    sg = (sg0, sg1)
    sw = (sw0, sw1)
    for t in range(NT):
        tab = tables[t]
        pltpu.sync_copy(ids[t].at[:, pl.ds(b0, CB)], idx_v)

        def gather(l, p, tab=tab):
            pltpu.async_copy(tab.at[idx_v.at[l]], rows[p], sg[p])

        def wait_gather(p, tab=tab):
            pltpu.make_async_copy(tab.at[idx_v.at[0]], rows[p],
                                  sg[p]).wait()

        def put(l, p, t=t):
            pltpu.async_copy(
                rows[p],
                out_hbm.at[l, pl.ds(b0, CB), pl.ds(t * DIM, DIM)], sw[p])

        def wait_put(p, t=t):
            pltpu.make_async_copy(
                rows[p],
                out_hbm.at[0, pl.ds(b0, CB), pl.ds(t * DIM, DIM)],
                sw[p]).wait()

        gather(0, 0)

        def body(l2, carry):
            for p in range(2):
                l = l2 * 2 + p
                pn = (p + 1) % 2

                @pl.when(l + 1 < L)
                def _(l=l, pn=pn):
                    @pl.when(l >= 1)
                    def _():
                        wait_put(pn)
                    gather(l + 1, pn)
                wait_gather(p)
                put(l, p)
            return carry

        lax.fori_loop(0, L // 2, body, 0)
        wait_put(0)
        wait_put(1)


@jax.jit
def _run(ids0, ids1, ids2, ids3, t0, t1, t2, t3):
    mesh = plsc.VectorSubcoreMesh(core_axis_name="c", subcore_axis_name="s",
                                  num_cores=NC, num_subcores=NS)
    k = pl.kernel(
        _emb_kernel,
        out_type=jax.ShapeDtypeStruct((L, B, NT * DIM), jnp.float32),
        mesh=mesh,
        scratch_types=(
            [pltpu.VMEM((L, CB), jnp.int32)]
            + [pltpu.VMEM((CB, DIM), jnp.float32)] * 2
            + [pltpu.SemaphoreType.DMA] * 4
        ),
        compiler_params=pltpu.CompilerParams(use_tc_tiling_on_sc=False),
    )
    return k(ids0, ids1, ids2, ids3, t0, t1, t2, t3)


def kernel(ids_0, ids_1, ids_2, ids_3, table_0, table_1, table_2, table_3):
    tabs = (table_0, table_1, table_2, table_3)
    tails = [tb[(v // CB) * CB:, :].reshape(-1)
             for tb, v in zip(tabs, VOCABS)]
    dense = _tr_run(table_0.T, table_1.T, table_2.T, table_3.T, *tails)
    tables = [d.reshape(v, DIM) for d, v in zip(dense, VOCABS)]
    ids = [i.astype(jnp.int32).T for i in (ids_0, ids_1, ids_2, ids_3)]
    out = _run(*ids, *tables)
    return out.transpose(1, 0, 2)


# speedup vs baseline: 3.8847x; 1.0467x over previous
import functools

import jax
import jax.numpy as jnp
from jax import lax
from jax.experimental import pallas as pl
from jax.experimental.pallas import tpu as pltpu
from jax.experimental.pallas import tpu_sc as plsc

NC = 2
NS = 16
NW = NC * NS

B = 4096
L = 50
DIM = 32
NT = 4
CB = B // NW
VL = 16
VOCABS = (1000000, 1000000, 100000, 100000)


def _tr_kernel(tt0, tt1, tt2, tt3, tl0, tl1, tl2, tl3, o0, o1, o2, o3,
               slab0, slab1, comb0, comb1, si0, si1, so0, so1):
    wid = lax.axis_index("s") * NC + lax.axis_index("c")
    slabs = (slab0, slab1)
    combs = (comb0, comb1)
    sin = (si0, si1)
    sout = (so0, so1)
    iota = lax.iota(jnp.int32, VL)
    idxc = [iota + u * VL for u in range(DIM // VL)]

    UNROLL = 8

    def transpose_rows(slab, comb, nrows):
        @plsc.parallel_loop(0, nrows, step=UNROLL, carry=iota)
        def rowbody(j, vi):
            pairs = []
            for r in range(UNROLL):
                vim = lax.bitwise_and(vi + r, CB - 1)
                st = lax.shift_left(vim, 5)
                for u in range(DIM // VL):
                    v = plsc.load_gather(slab, [idxc[u], vim])
                    pairs.append((lax.bitwise_or(st, idxc[u]), v))
            for sidx, v in pairs:
                plsc.store_scatter(comb, [sidx], v)
            return vi + UNROLL

    for t, (tt, tl, out) in enumerate(zip((tt0, tt1, tt2, tt3),
                                          (tl0, tl1, tl2, tl3),
                                          (o0, o1, o2, o3))):
        v = VOCABS[t]
        nfull = v // CB
        tail = v % CB
        nj = (nfull - wid + NW - 1) // NW

        def issue_in(k, p, tt=tt, nj=nj):
            @pl.when(k < nj)
            def _():
                blk = wid + k * NW
                pltpu.async_copy(tt.at[:, pl.ds(blk * CB, CB)],
                                 slabs[p], sin[p])

        def wait_in(p, tt=tt):
            pltpu.make_async_copy(tt.at[:, pl.ds(0, CB)], slabs[p],
                                  sin[p]).wait()

        def issue_out(k, p, out=out):
            blk = wid + k * NW
            pltpu.async_copy(combs[p],
                             out.at[pl.ds(blk * CB * DIM, CB * DIM)],
                             sout[p])

        def wait_out(p, out=out):
            pltpu.make_async_copy(out.at[pl.ds(0, CB * DIM)], combs[p],
                                  sout[p]).wait()

        issue_in(0, 0)
        issue_in(1, 1)

        def pairbody(k2, c, nj=nj):
            for p in range(2):
                k = k2 * 2 + p

                @pl.when(k < nj)
                def _(k=k, p=p):
                    wait_in(p)

                    @pl.when(k >= 2)
                    def _():
                        wait_out(p)
                    transpose_rows(slabs[p], combs[p], CB)
                    issue_out(k, p)
                    issue_in(k + 2, p)
            return c

        lax.fori_loop(0, (nj + 1) // 2, pairbody, 0)

        @pl.when(nj >= 1)
        def _():
            wait_out(0)

        @pl.when(nj >= 2)
        def _():
            wait_out(1)

        if tail:
            owner = nfull % NW

            @pl.when(wid == owner)
            def _(tl=tl, out=out, nfull=nfull, tail=tail):
                pltpu.sync_copy(
                    tl, out.at[pl.ds(nfull * CB * DIM, tail * DIM)])


@jax.jit
def _tr_run(tt0, tt1, tt2, tt3, tl0, tl1, tl2, tl3):
    mesh = plsc.VectorSubcoreMesh(core_axis_name="c", subcore_axis_name="s",
                                  num_cores=NC, num_subcores=NS)
    k = pl.kernel(
        _tr_kernel,
        out_type=tuple(jax.ShapeDtypeStruct((v * DIM,), jnp.float32)
                       for v in VOCABS),
        mesh=mesh,
        scratch_types=(
            [pltpu.VMEM((DIM, CB), jnp.float32)] * 2
            + [pltpu.VMEM((CB * DIM,), jnp.float32)] * 2
            + [pltpu.SemaphoreType.DMA] * 4
        ),
        compiler_params=pltpu.CompilerParams(use_tc_tiling_on_sc=True,
                                             needs_layout_passes=False),
    )
    return k(tt0, tt1, tt2, tt3, tl0, tl1, tl2, tl3)


def _emb_kernel(ids0, ids1, ids2, ids3, t0, t1, t2, t3, out_hbm,
                idx_v, rows0, rows1, sg0, sg1, sw0, sw1):
    wid = lax.axis_index("s") * NC + lax.axis_index("c")
    b0 = wid * CB
    tables = (t0, t1, t2, t3)
    ids = (ids0, ids1, ids2, ids3)
    rows = (rows0, rows1)
    sg = (sg0, sg1)
    sw = (sw0, sw1)
    for t in range(NT):
        tab = tables[t]
        pltpu.sync_copy(ids[t].at[:, pl.ds(b0, CB)], idx_v)

        def gather(l, p, tab=tab):
            pltpu.async_copy(tab.at[idx_v.at[l]], rows[p], sg[p])

        def wait_gather(p, tab=tab):
            pltpu.make_async_copy(tab.at[idx_v.at[0]], rows[p],
                                  sg[p]).wait()

        def put(l, p, t=t):
            pltpu.async_copy(
                rows[p],
                out_hbm.at[l, pl.ds(b0, CB), pl.ds(t * DIM, DIM)], sw[p])

        def wait_put(p, t=t):
            pltpu.make_async_copy(
                rows[p],
                out_hbm.at[0, pl.ds(b0, CB), pl.ds(t * DIM, DIM)],
                sw[p]).wait()

        gather(0, 0)

        def body(l2, carry):
            for p in range(2):
                l = l2 * 2 + p
                pn = (p + 1) % 2

                @pl.when(l + 1 < L)
                def _(l=l, pn=pn):
                    @pl.when(l >= 1)
                    def _():
                        wait_put(pn)
                    gather(l + 1, pn)
                wait_gather(p)
                put(l, p)
            return carry

        lax.fori_loop(0, L // 2, body, 0)
        wait_put(0)
        wait_put(1)


@jax.jit
def _run(ids0, ids1, ids2, ids3, t0, t1, t2, t3):
    mesh = plsc.VectorSubcoreMesh(core_axis_name="c", subcore_axis_name="s",
                                  num_cores=NC, num_subcores=NS)
    k = pl.kernel(
        _emb_kernel,
        out_type=jax.ShapeDtypeStruct((L, B, NT * DIM), jnp.float32),
        mesh=mesh,
        scratch_types=(
            [pltpu.VMEM((L, CB), jnp.int32)]
            + [pltpu.VMEM((CB, DIM), jnp.float32)] * 2
            + [pltpu.SemaphoreType.DMA] * 4
        ),
        compiler_params=pltpu.CompilerParams(use_tc_tiling_on_sc=False),
    )
    return k(ids0, ids1, ids2, ids3, t0, t1, t2, t3)


def kernel(ids_0, ids_1, ids_2, ids_3, table_0, table_1, table_2, table_3):
    tabs = (table_0, table_1, table_2, table_3)
    tails = [tb[(v // CB) * CB:, :].reshape(-1)
             for tb, v in zip(tabs, VOCABS)]
    dense = _tr_run(table_0.T, table_1.T, table_2.T, table_3.T, *tails)
    tables = [d.reshape(v, DIM) for d, v in zip(dense, VOCABS)]
    ids = [i.astype(jnp.int32).T for i in (ids_0, ids_1, ids_2, ids_3)]
    out = _run(*ids, *tables)
    return out.transpose(1, 0, 2)


# speedup vs baseline: 4.6102x; 1.1868x over previous
import functools

import jax
import jax.numpy as jnp
from jax import lax
from jax.experimental import pallas as pl
from jax.experimental.pallas import tpu as pltpu
from jax.experimental.pallas import tpu_sc as plsc

NC = 2
NS = 16
NW = NC * NS

B = 4096
L = 50
DIM = 32
NT = 4
CB = B // NW
SB = 256
VL = 16
VOCABS = (1000000, 1000000, 100000, 100000)


def _tr_kernel(tt0, tt1, tt2, tt3, tl0, tl1, tl2, tl3, o0, o1, o2, o3,
               slab0, slab1, comb0, comb1, si0, si1, so0, so1):
    wid = lax.axis_index("s") * NC + lax.axis_index("c")
    slabs = (slab0, slab1)
    combs = (comb0, comb1)
    sin = (si0, si1)
    sout = (so0, so1)
    iota = lax.iota(jnp.int32, VL)
    idxc = [iota + u * VL for u in range(DIM // VL)]

    UNROLL = 8

    def transpose_rows(slab, comb, nrows):
        @plsc.parallel_loop(0, nrows, step=UNROLL, carry=iota)
        def rowbody(j, vi):
            pairs = []
            for r in range(UNROLL):
                vim = lax.bitwise_and(vi + r, SB - 1)
                st = lax.shift_left(vim, 5)
                for u in range(DIM // VL):
                    v = plsc.load_gather(slab, [idxc[u], vim])
                    pairs.append((lax.bitwise_or(st, idxc[u]), v))
            for sidx, v in pairs:
                plsc.store_scatter(comb, [sidx], v)
            return vi + UNROLL

    for t, (tt, tl, out) in enumerate(zip((tt0, tt1, tt2, tt3),
                                          (tl0, tl1, tl2, tl3),
                                          (o0, o1, o2, o3))):
        v = VOCABS[t]
        nfull = v // SB
        tail = v % SB
        nj = (nfull - wid + NW - 1) // NW

        def issue_in(k, p, tt=tt, nj=nj):
            @pl.when(k < nj)
            def _():
                blk = wid + k * NW
                pltpu.async_copy(tt.at[:, pl.ds(blk * SB, SB)],
                                 slabs[p], sin[p])

        def wait_in(p, tt=tt):
            pltpu.make_async_copy(tt.at[:, pl.ds(0, SB)], slabs[p],
                                  sin[p]).wait()

        def issue_out(k, p, out=out):
            blk = wid + k * NW
            pltpu.async_copy(combs[p],
                             out.at[pl.ds(blk * SB * DIM, SB * DIM)],
                             sout[p])

        def wait_out(p, out=out):
            pltpu.make_async_copy(out.at[pl.ds(0, SB * DIM)], combs[p],
                                  sout[p]).wait()

        issue_in(0, 0)
        issue_in(1, 1)

        def pairbody(k2, c, nj=nj):
            for p in range(2):
                k = k2 * 2 + p

                @pl.when(k < nj)
                def _(k=k, p=p):
                    wait_in(p)

                    @pl.when(k >= 2)
                    def _():
                        wait_out(p)
                    transpose_rows(slabs[p], combs[p], SB)
                    issue_out(k, p)
                    issue_in(k + 2, p)
            return c

        lax.fori_loop(0, (nj + 1) // 2, pairbody, 0)

        @pl.when(nj >= 1)
        def _():
            wait_out(0)

        @pl.when(nj >= 2)
        def _():
            wait_out(1)

        if tail:
            owner = nfull % NW

            @pl.when(wid == owner)
            def _(tl=tl, out=out, nfull=nfull, tail=tail):
                pltpu.sync_copy(
                    tl, out.at[pl.ds(nfull * SB * DIM, tail * DIM)])


@jax.jit
def _tr_run(tt0, tt1, tt2, tt3, tl0, tl1, tl2, tl3):
    mesh = plsc.VectorSubcoreMesh(core_axis_name="c", subcore_axis_name="s",
                                  num_cores=NC, num_subcores=NS)
    k = pl.kernel(
        _tr_kernel,
        out_type=tuple(jax.ShapeDtypeStruct((v * DIM,), jnp.float32)
                       for v in VOCABS),
        mesh=mesh,
        scratch_types=(
            [pltpu.VMEM((DIM, SB), jnp.float32)] * 2
            + [pltpu.VMEM((SB * DIM,), jnp.float32)] * 2
            + [pltpu.SemaphoreType.DMA] * 4
        ),
        compiler_params=pltpu.CompilerParams(use_tc_tiling_on_sc=True,
                                             needs_layout_passes=False),
    )
    return k(tt0, tt1, tt2, tt3, tl0, tl1, tl2, tl3)


def _emb_kernel(ids0, ids1, ids2, ids3, t0, t1, t2, t3, out_hbm,
                idx_v, rows0, rows1, sg0, sg1, sw0, sw1):
    wid = lax.axis_index("s") * NC + lax.axis_index("c")
    b0 = wid * CB
    tables = (t0, t1, t2, t3)
    ids = (ids0, ids1, ids2, ids3)
    rows = (rows0, rows1)
    sg = (sg0, sg1)
    sw = (sw0, sw1)
    for t in range(NT):
        tab = tables[t]
        pltpu.sync_copy(ids[t].at[:, pl.ds(b0, CB)], idx_v)

        def gather(l, p, tab=tab):
            pltpu.async_copy(tab.at[idx_v.at[l]], rows[p], sg[p])

        def wait_gather(p, tab=tab):
            pltpu.make_async_copy(tab.at[idx_v.at[0]], rows[p],
                                  sg[p]).wait()

        def put(l, p, t=t):
            pltpu.async_copy(
                rows[p],
                out_hbm.at[l, pl.ds(b0, CB), pl.ds(t * DIM, DIM)], sw[p])

        def wait_put(p, t=t):
            pltpu.make_async_copy(
                rows[p],
                out_hbm.at[0, pl.ds(b0, CB), pl.ds(t * DIM, DIM)],
                sw[p]).wait()

        gather(0, 0)

        def body(l2, carry):
            for p in range(2):
                l = l2 * 2 + p
                pn = (p + 1) % 2

                @pl.when(l + 1 < L)
                def _(l=l, pn=pn):
                    @pl.when(l >= 1)
                    def _():
                        wait_put(pn)
                    gather(l + 1, pn)
                wait_gather(p)
                put(l, p)
            return carry

        lax.fori_loop(0, L // 2, body, 0)
        wait_put(0)
        wait_put(1)


@jax.jit
def _run(ids0, ids1, ids2, ids3, t0, t1, t2, t3):
    mesh = plsc.VectorSubcoreMesh(core_axis_name="c", subcore_axis_name="s",
                                  num_cores=NC, num_subcores=NS)
    k = pl.kernel(
        _emb_kernel,
        out_type=jax.ShapeDtypeStruct((L, B, NT * DIM), jnp.float32),
        mesh=mesh,
        scratch_types=(
            [pltpu.VMEM((L, CB), jnp.int32)]
            + [pltpu.VMEM((CB, DIM), jnp.float32)] * 2
            + [pltpu.SemaphoreType.DMA] * 4
        ),
        compiler_params=pltpu.CompilerParams(use_tc_tiling_on_sc=False),
    )
    return k(ids0, ids1, ids2, ids3, t0, t1, t2, t3)


def kernel(ids_0, ids_1, ids_2, ids_3, table_0, table_1, table_2, table_3):
    tabs = (table_0, table_1, table_2, table_3)
    tails = [tb[(v // SB) * SB:, :].reshape(-1)
             for tb, v in zip(tabs, VOCABS)]
    dense = _tr_run(table_0.T, table_1.T, table_2.T, table_3.T, *tails)
    tables = [d.reshape(v, DIM) for d, v in zip(dense, VOCABS)]
    ids = [i.astype(jnp.int32).T for i in (ids_0, ids_1, ids_2, ids_3)]
    out = _run(*ids, *tables)
    return out.transpose(1, 0, 2)


# speedup vs baseline: 5.1632x; 1.1200x over previous
import functools

import jax
import jax.numpy as jnp
from jax import lax
from jax.experimental import pallas as pl
from jax.experimental.pallas import tpu as pltpu
from jax.experimental.pallas import tpu_sc as plsc

NC = 2
NS = 16
NW = NC * NS

B = 4096
L = 50
DIM = 32
NT = 4
CB = B // NW
SB = 512
VL = 16
VOCABS = (1000000, 1000000, 100000, 100000)


def _tr_kernel(tt0, tt1, tt2, tt3, tl0, tl1, tl2, tl3, o0, o1, o2, o3,
               slab0, slab1, comb0, comb1, si0, si1, so0, so1):
    wid = lax.axis_index("s") * NC + lax.axis_index("c")
    slabs = (slab0, slab1)
    combs = (comb0, comb1)
    sin = (si0, si1)
    sout = (so0, so1)
    iota = lax.iota(jnp.int32, VL)
    idxc = [iota + u * VL for u in range(DIM // VL)]

    UNROLL = 8

    def transpose_rows(slab, comb, nrows):
        @plsc.parallel_loop(0, nrows, step=UNROLL, carry=iota)
        def rowbody(j, vi):
            pairs = []
            for r in range(UNROLL):
                vim = lax.bitwise_and(vi + r, SB - 1)
                st = lax.shift_left(vim, 5)
                for u in range(DIM // VL):
                    v = plsc.load_gather(slab, [idxc[u], vim])
                    pairs.append((lax.bitwise_or(st, idxc[u]), v))
            for sidx, v in pairs:
                plsc.store_scatter(comb, [sidx], v)
            return vi + UNROLL

    for t, (tt, tl, out) in enumerate(zip((tt0, tt1, tt2, tt3),
                                          (tl0, tl1, tl2, tl3),
                                          (o0, o1, o2, o3))):
        v = VOCABS[t]
        nfull = v // SB
        tail = v % SB
        nj = (nfull - wid + NW - 1) // NW

        def issue_in(k, p, tt=tt, nj=nj):
            @pl.when(k < nj)
            def _():
                blk = wid + k * NW
                pltpu.async_copy(tt.at[:, pl.ds(blk * SB, SB)],
                                 slabs[p], sin[p])

        def wait_in(p, tt=tt):
            pltpu.make_async_copy(tt.at[:, pl.ds(0, SB)], slabs[p],
                                  sin[p]).wait()

        def issue_out(k, p, out=out):
            blk = wid + k * NW
            pltpu.async_copy(combs[p],
                             out.at[pl.ds(blk * SB * DIM, SB * DIM)],
                             sout[p])

        def wait_out(p, out=out):
            pltpu.make_async_copy(out.at[pl.ds(0, SB * DIM)], combs[p],
                                  sout[p]).wait()

        issue_in(0, 0)
        issue_in(1, 1)

        def pairbody(k2, c, nj=nj):
            for p in range(2):
                k = k2 * 2 + p

                @pl.when(k < nj)
                def _(k=k, p=p):
                    wait_in(p)

                    @pl.when(k >= 2)
                    def _():
                        wait_out(p)
                    transpose_rows(slabs[p], combs[p], SB)
                    issue_out(k, p)
                    issue_in(k + 2, p)
            return c

        lax.fori_loop(0, (nj + 1) // 2, pairbody, 0)

        @pl.when(nj >= 1)
        def _():
            wait_out(0)

        @pl.when(nj >= 2)
        def _():
            wait_out(1)

        if tail:
            owner = nfull % NW

            @pl.when(wid == owner)
            def _(tl=tl, out=out, nfull=nfull, tail=tail):
                pltpu.sync_copy(
                    tl, out.at[pl.ds(nfull * SB * DIM, tail * DIM)])


@jax.jit
def _tr_run(tt0, tt1, tt2, tt3, tl0, tl1, tl2, tl3):
    mesh = plsc.VectorSubcoreMesh(core_axis_name="c", subcore_axis_name="s",
                                  num_cores=NC, num_subcores=NS)
    k = pl.kernel(
        _tr_kernel,
        out_type=tuple(jax.ShapeDtypeStruct((v * DIM,), jnp.float32)
                       for v in VOCABS),
        mesh=mesh,
        scratch_types=(
            [pltpu.VMEM((DIM, SB), jnp.float32)] * 2
            + [pltpu.VMEM((SB * DIM,), jnp.float32)] * 2
            + [pltpu.SemaphoreType.DMA] * 4
        ),
        compiler_params=pltpu.CompilerParams(use_tc_tiling_on_sc=True,
                                             needs_layout_passes=False),
    )
    return k(tt0, tt1, tt2, tt3, tl0, tl1, tl2, tl3)


def _emb_kernel(ids0, ids1, ids2, ids3, t0, t1, t2, t3, out_hbm,
                idx_v, rows0, rows1, sg0, sg1, sw0, sw1):
    wid = lax.axis_index("s") * NC + lax.axis_index("c")
    b0 = wid * CB
    tables = (t0, t1, t2, t3)
    ids = (ids0, ids1, ids2, ids3)
    rows = (rows0, rows1)
    sg = (sg0, sg1)
    sw = (sw0, sw1)
    for t in range(NT):
        tab = tables[t]
        pltpu.sync_copy(ids[t].at[:, pl.ds(b0, CB)], idx_v)

        def gather(l, p, tab=tab):
            pltpu.async_copy(tab.at[idx_v.at[l]], rows[p], sg[p])

        def wait_gather(p, tab=tab):
            pltpu.make_async_copy(tab.at[idx_v.at[0]], rows[p],
                                  sg[p]).wait()

        def put(l, p, t=t):
            pltpu.async_copy(
                rows[p],
                out_hbm.at[l, pl.ds(b0, CB), pl.ds(t * DIM, DIM)], sw[p])

        def wait_put(p, t=t):
            pltpu.make_async_copy(
                rows[p],
                out_hbm.at[0, pl.ds(b0, CB), pl.ds(t * DIM, DIM)],
                sw[p]).wait()

        gather(0, 0)

        def body(l2, carry):
            for p in range(2):
                l = l2 * 2 + p
                pn = (p + 1) % 2

                @pl.when(l + 1 < L)
                def _(l=l, pn=pn):
                    @pl.when(l >= 1)
                    def _():
                        wait_put(pn)
                    gather(l + 1, pn)
                wait_gather(p)
                put(l, p)
            return carry

        lax.fori_loop(0, L // 2, body, 0)
        wait_put(0)
        wait_put(1)


@jax.jit
def _run(ids0, ids1, ids2, ids3, t0, t1, t2, t3):
    mesh = plsc.VectorSubcoreMesh(core_axis_name="c", subcore_axis_name="s",
                                  num_cores=NC, num_subcores=NS)
    k = pl.kernel(
        _emb_kernel,
        out_type=jax.ShapeDtypeStruct((L, B, NT * DIM), jnp.float32),
        mesh=mesh,
        scratch_types=(
            [pltpu.VMEM((L, CB), jnp.int32)]
            + [pltpu.VMEM((CB, DIM), jnp.float32)] * 2
            + [pltpu.SemaphoreType.DMA] * 4
        ),
        compiler_params=pltpu.CompilerParams(use_tc_tiling_on_sc=False),
    )
    return k(ids0, ids1, ids2, ids3, t0, t1, t2, t3)


def kernel(ids_0, ids_1, ids_2, ids_3, table_0, table_1, table_2, table_3):
    tabs = (table_0, table_1, table_2, table_3)
    tails = [tb[(v // SB) * SB:, :].reshape(-1)
             for tb, v in zip(tabs, VOCABS)]
    dense = _tr_run(table_0.T, table_1.T, table_2.T, table_3.T, *tails)
    tables = [d.reshape(v, DIM) for d, v in zip(dense, VOCABS)]
    ids = [i.astype(jnp.int32).T for i in (ids_0, ids_1, ids_2, ids_3)]
    out = _run(*ids, *tables)
    return out.transpose(1, 0, 2)


# speedup vs baseline: 5.3299x; 1.0323x over previous
import functools

import jax
import jax.numpy as jnp
from jax import lax
from jax.experimental import pallas as pl
from jax.experimental.pallas import tpu as pltpu
from jax.experimental.pallas import tpu_sc as plsc

NC = 2
NS = 16
NW = NC * NS

B = 4096
L = 50
DIM = 32
NT = 4
CB = B // NW
SB = 768
VL = 16
VOCABS = (1000000, 1000000, 100000, 100000)


def _tr_kernel(tt0, tt1, tt2, tt3, tl0, tl1, tl2, tl3, o0, o1, o2, o3,
               slab0, slab1, comb0, comb1, si0, si1, so0, so1):
    wid = lax.axis_index("s") * NC + lax.axis_index("c")
    slabs = (slab0, slab1)
    combs = (comb0, comb1)
    sin = (si0, si1)
    sout = (so0, so1)
    iota = lax.iota(jnp.int32, VL)
    idxc = [iota + u * VL for u in range(DIM // VL)]

    UNROLL = 8

    def transpose_rows(slab, comb, nrows):
        @plsc.parallel_loop(0, nrows, step=UNROLL, carry=iota)
        def rowbody(j, vi):
            pairs = []
            for r in range(UNROLL):
                vim = lax.bitwise_and(vi + r, SB - 1)
                st = lax.shift_left(vim, 5)
                for u in range(DIM // VL):
                    v = plsc.load_gather(slab, [idxc[u], vim])
                    pairs.append((lax.bitwise_or(st, idxc[u]), v))
            for sidx, v in pairs:
                plsc.store_scatter(comb, [sidx], v)
            return vi + UNROLL

    for t, (tt, tl, out) in enumerate(zip((tt0, tt1, tt2, tt3),
                                          (tl0, tl1, tl2, tl3),
                                          (o0, o1, o2, o3))):
        v = VOCABS[t]
        nfull = v // SB
        tail = v % SB
        nj = (nfull - wid + NW - 1) // NW

        def issue_in(k, p, tt=tt, nj=nj):
            @pl.when(k < nj)
            def _():
                blk = wid + k * NW
                pltpu.async_copy(tt.at[:, pl.ds(blk * SB, SB)],
                                 slabs[p], sin[p])

        def wait_in(p, tt=tt):
            pltpu.make_async_copy(tt.at[:, pl.ds(0, SB)], slabs[p],
                                  sin[p]).wait()

        def issue_out(k, p, out=out):
            blk = wid + k * NW
            pltpu.async_copy(combs[p],
                             out.at[pl.ds(blk * SB * DIM, SB * DIM)],
                             sout[p])

        def wait_out(p, out=out):
            pltpu.make_async_copy(out.at[pl.ds(0, SB * DIM)], combs[p],
                                  sout[p]).wait()

        issue_in(0, 0)
        issue_in(1, 1)

        def pairbody(k2, c, nj=nj):
            for p in range(2):
                k = k2 * 2 + p

                @pl.when(k < nj)
                def _(k=k, p=p):
                    wait_in(p)

                    @pl.when(k >= 2)
                    def _():
                        wait_out(p)
                    transpose_rows(slabs[p], combs[p], SB)
                    issue_out(k, p)
                    issue_in(k + 2, p)
            return c

        lax.fori_loop(0, (nj + 1) // 2, pairbody, 0)

        @pl.when(nj >= 1)
        def _():
            wait_out(0)

        @pl.when(nj >= 2)
        def _():
            wait_out(1)

        if tail:
            owner = nfull % NW

            @pl.when(wid == owner)
            def _(tl=tl, out=out, nfull=nfull, tail=tail):
                pltpu.sync_copy(
                    tl, out.at[pl.ds(nfull * SB * DIM, tail * DIM)])


@jax.jit
def _tr_run(tt0, tt1, tt2, tt3, tl0, tl1, tl2, tl3):
    mesh = plsc.VectorSubcoreMesh(core_axis_name="c", subcore_axis_name="s",
                                  num_cores=NC, num_subcores=NS)
    k = pl.kernel(
        _tr_kernel,
        out_type=tuple(jax.ShapeDtypeStruct((v * DIM,), jnp.float32)
                       for v in VOCABS),
        mesh=mesh,
        scratch_types=(
            [pltpu.VMEM((DIM, SB), jnp.float32)] * 2
            + [pltpu.VMEM((SB * DIM,), jnp.float32)] * 2
            + [pltpu.SemaphoreType.DMA] * 4
        ),
        compiler_params=pltpu.CompilerParams(use_tc_tiling_on_sc=True,
                                             needs_layout_passes=False),
    )
    return k(tt0, tt1, tt2, tt3, tl0, tl1, tl2, tl3)


def _emb_kernel(ids0, ids1, ids2, ids3, t0, t1, t2, t3, out_hbm,
                idx_v, rows0, rows1, sg0, sg1, sw0, sw1):
    wid = lax.axis_index("s") * NC + lax.axis_index("c")
    b0 = wid * CB
    tables = (t0, t1, t2, t3)
    ids = (ids0, ids1, ids2, ids3)
    rows = (rows0, rows1)
    sg = (sg0, sg1)
    sw = (sw0, sw1)
    for t in range(NT):
        tab = tables[t]
        pltpu.sync_copy(ids[t].at[:, pl.ds(b0, CB)], idx_v)

        def gather(l, p, tab=tab):
            pltpu.async_copy(tab.at[idx_v.at[l]], rows[p], sg[p])

        def wait_gather(p, tab=tab):
            pltpu.make_async_copy(tab.at[idx_v.at[0]], rows[p],
                                  sg[p]).wait()

        def put(l, p, t=t):
            pltpu.async_copy(
                rows[p],
                out_hbm.at[l, pl.ds(b0, CB), pl.ds(t * DIM, DIM)], sw[p])

        def wait_put(p, t=t):
            pltpu.make_async_copy(
                rows[p],
                out_hbm.at[0, pl.ds(b0, CB), pl.ds(t * DIM, DIM)],
                sw[p]).wait()

        gather(0, 0)

        def body(l2, carry):
            for p in range(2):
                l = l2 * 2 + p
                pn = (p + 1) % 2

                @pl.when(l + 1 < L)
                def _(l=l, pn=pn):
                    @pl.when(l >= 1)
                    def _():
                        wait_put(pn)
                    gather(l + 1, pn)
                wait_gather(p)
                put(l, p)
            return carry

        lax.fori_loop(0, L // 2, body, 0)
        wait_put(0)
        wait_put(1)


@jax.jit
def _run(ids0, ids1, ids2, ids3, t0, t1, t2, t3):
    mesh = plsc.VectorSubcoreMesh(core_axis_name="c", subcore_axis_name="s",
                                  num_cores=NC, num_subcores=NS)
    k = pl.kernel(
        _emb_kernel,
        out_type=jax.ShapeDtypeStruct((L, B, NT * DIM), jnp.float32),
        mesh=mesh,
        scratch_types=(
            [pltpu.VMEM((L, CB), jnp.int32)]
            + [pltpu.VMEM((CB, DIM), jnp.float32)] * 2
            + [pltpu.SemaphoreType.DMA] * 4
        ),
        compiler_params=pltpu.CompilerParams(use_tc_tiling_on_sc=False),
    )
    return k(ids0, ids1, ids2, ids3, t0, t1, t2, t3)


def kernel(ids_0, ids_1, ids_2, ids_3, table_0, table_1, table_2, table_3):
    tabs = (table_0, table_1, table_2, table_3)
    tails = [tb[(v // SB) * SB:, :].reshape(-1)
             for tb, v in zip(tabs, VOCABS)]
    dense = _tr_run(table_0.T, table_1.T, table_2.T, table_3.T, *tails)
    tables = [d.reshape(v, DIM) for d, v in zip(dense, VOCABS)]
    ids = [i.astype(jnp.int32).T for i in (ids_0, ids_1, ids_2, ids_3)]
    out = _run(*ids, *tables)
    return out.transpose(1, 0, 2)
